# baseline (speedup 1.0000x reference)
import functools

import jax
import jax.numpy as jnp
from jax import lax
from jax.experimental import pallas as pl
from jax.experimental.pallas import tpu as pltpu
from jax.experimental.pallas import tpu_sc as plsc

_NUM_SAMPLES = 32


def kernel(x):
    C, T, H, W = x.shape
    n = _NUM_SAMPLES
    rows = C * n
    nw = 32
    per_w = rows // nw
    assert per_w * nw == rows

    mesh = plsc.VectorSubcoreMesh(core_axis_name="c", subcore_axis_name="s")

    @functools.partial(
        pl.kernel,
        mesh=mesh,
        out_type=jax.ShapeDtypeStruct((C, n, H, W), x.dtype),
        scratch_types=[pltpu.SemaphoreType.DMA],
    )
    def k(x_hbm, out_hbm, sem):
        wid = lax.axis_index("s") * 2 + lax.axis_index("c")
        copies = []
        for j in range(per_w):
            r = wid * per_w + j
            c = r // n
            t = r % n
            tsrc = (t * (T - 1)) // (n - 1)
            copies.append(
                pltpu.make_async_copy(x_hbm.at[c, tsrc], out_hbm.at[c, t], sem)
            )
        for cp in copies:
            cp.start()
        for cp in copies:
            cp.wait()

    return k(x)


# speedup vs baseline: 5.6405x; 5.6405x over previous
import functools

import jax
import jax.numpy as jnp
from jax import lax
from jax.experimental import pallas as pl
from jax.experimental.pallas import tpu as pltpu
from jax.experimental.pallas import tpu_sc as plsc

_NUM_SAMPLES = 32


def kernel(x):
    C, T, H, W = x.shape
    n = _NUM_SAMPLES
    HW = H * W
    CH = 32768
    nch = HW // CH
    nbuf = 3
    rows = C * n
    nw = 32
    per_w = rows // nw
    total = per_w * nch
    assert per_w * nw == rows and nch * CH == HW

    x1 = x.reshape(C * T * HW)
    mesh = plsc.VectorSubcoreMesh(core_axis_name="c", subcore_axis_name="s")

    @functools.partial(
        pl.kernel,
        mesh=mesh,
        out_type=jax.ShapeDtypeStruct((rows * HW,), x.dtype),
        scratch_types=[pltpu.VMEM((CH,), x.dtype) for _ in range(nbuf)]
        + [pltpu.SemaphoreType.DMA] * (2 * nbuf),
    )
    def k(x_hbm, out_hbm, *scratch):
        vbuf = scratch[:nbuf]
        gsem, ssem = scratch[nbuf : 2 * nbuf], scratch[2 * nbuf :]
        wid = lax.axis_index("s") * 2 + lax.axis_index("c")

        def src_dst(q):
            frame, ch = q // nch, q % nch
            r = wid * per_w + frame
            c = r // n
            t = r % n
            tsrc = (t * (T - 1)) // (n - 1)
            src = x_hbm.at[pl.ds((c * T + tsrc) * HW + ch * CH, CH)]
            dst = out_hbm.at[pl.ds(r * HW + ch * CH, CH)]
            return src, dst

        def start_gather(q, b):
            src, _ = src_dst(q)
            pltpu.make_async_copy(src, vbuf[b], gsem[b]).start()

        for q in range(min(nbuf, total)):
            start_gather(q, q % nbuf)
        for q in range(total):
            b = q % nbuf
            src, dst = src_dst(q)
            pltpu.make_async_copy(src, vbuf[b], gsem[b]).wait()
            scat = pltpu.make_async_copy(vbuf[b], dst, ssem[b])
            scat.start()
            if q + nbuf < total:
                scat.wait()
                start_gather(q + nbuf, b)
        for q in range(max(0, total - nbuf), total):
            b = q % nbuf
            _, dst = src_dst(q)
            pltpu.make_async_copy(vbuf[b], dst, ssem[b]).wait()

    return k(x1).reshape(C, n, H, W)


# speedup vs baseline: 21.2202x; 3.7621x over previous
import functools

import jax
import jax.numpy as jnp
from jax import lax
from jax.experimental import pallas as pl
from jax.experimental.pallas import tpu as pltpu
from jax.experimental.pallas import tpu_sc as plsc

_NUM_SAMPLES = 32


def kernel(x):
    C, T, H, W = x.shape
    n = _NUM_SAMPLES
    hch = 128
    nch = H // hch
    nbuf = 3
    rows = C * n
    nw = 32
    per_w = rows // nw
    total = per_w * nch
    assert per_w * nw == rows and nch * hch == H

    mesh = plsc.VectorSubcoreMesh(core_axis_name="c", subcore_axis_name="s")

    x2 = x.reshape(C * T * H, W)

    @functools.partial(
        pl.kernel,
        mesh=mesh,
        out_type=jax.ShapeDtypeStruct((rows * H, W), x.dtype),
        scratch_types=[pltpu.VMEM((hch, W), x.dtype) for _ in range(nbuf)]
        + [pltpu.SemaphoreType.DMA] * (2 * nbuf),
    )
    def k(x_hbm, out_hbm, *scratch):
        vbuf = scratch[:nbuf]
        gsem, ssem = scratch[nbuf : 2 * nbuf], scratch[2 * nbuf :]
        wid = lax.axis_index("s") * 2 + lax.axis_index("c")

        def src_dst(q):
            frame, ch = q // nch, q % nch
            r = wid * per_w + frame
            c = r // n
            t = r % n
            tsrc = (t * (T - 1)) // (n - 1)
            src = x_hbm.at[pl.ds((c * T + tsrc) * H + ch * hch, hch), :]
            dst = out_hbm.at[pl.ds(r * H + ch * hch, hch), :]
            return src, dst

        def start_gather(q, b):
            src, _ = src_dst(q)
            pltpu.make_async_copy(src, vbuf[b], gsem[b]).start()

        for q in range(min(nbuf, total)):
            start_gather(q, q % nbuf)
        for q in range(total):
            b = q % nbuf
            src, dst = src_dst(q)
            pltpu.make_async_copy(src, vbuf[b], gsem[b]).wait()
            scat = pltpu.make_async_copy(vbuf[b], dst, ssem[b])
            scat.start()
            if q + nbuf < total:
                scat.wait()
                start_gather(q + nbuf, b)
        for q in range(max(0, total - nbuf), total):
            b = q % nbuf
            _, dst = src_dst(q)
            pltpu.make_async_copy(vbuf[b], dst, ssem[b]).wait()

    return k(x2).reshape(C, n, H, W)
